# baseline (speedup 1.0000x reference)
import functools

import jax
import jax.numpy as jnp
from jax import lax
from jax.experimental import pallas as pl
from jax.experimental.pallas import tpu as pltpu
from jax.experimental.pallas import tpu_sc as plsc

LANES = 16
SUB = 128
BATCH = 512
NCORES = 2
NTILES = 16
BN = 3584


def _leaky(x):
    return jnp.where(x >= 0, x, 0.1 * x)


def _sc_deg_body(np_, nsub_pt, col, ew, out, dacc, colb, ewb, zb):
    cid = lax.axis_index("c")
    sid = lax.axis_index("s")
    npt = np_ // NTILES
    sub_pb = BATCH // SUB

    @pl.loop(0, npt // LANES)
    def _(i):
        zb[pl.ds(i * LANES, LANES)] = jnp.zeros((LANES,), jnp.float32)

    pltpu.sync_copy(zb, dacc.at[pl.ds(sid * npt, npt)])
    plsc.subcore_barrier()

    wid = cid * NTILES + sid
    nb = nsub_pt // sub_pb

    @pl.loop(0, nb)
    def _(b):
        eb = wid * (nsub_pt * SUB) + b * BATCH
        for j in range(sub_pb):
            pltpu.sync_copy(col.at[pl.ds(eb + j * SUB, SUB)], colb.at[j])
        pltpu.sync_copy(ew.at[pl.ds(eb, BATCH)], ewb)
        for j in range(sub_pb):
            pltpu.sync_copy(ewb.at[pl.ds(j * SUB, SUB)],
                            dacc.at[colb.at[j]], add=True)

    plsc.subcore_barrier()
    pltpu.sync_copy(dacc.at[pl.ds(sid * npt, npt)],
                    out.at[cid].at[pl.ds(sid * npt, npt)])


def _sc_deg(col, ew, np_, ep):
    nsub_pt = ep // (NCORES * NTILES * SUB)
    mesh = plsc.VectorSubcoreMesh(core_axis_name="c", subcore_axis_name="s")
    return pl.kernel(
        functools.partial(_sc_deg_body, np_, nsub_pt),
        out_type=jax.ShapeDtypeStruct((NCORES, np_), jnp.float32),
        mesh=mesh,
        scratch_types=[
            pltpu.VMEM_SHARED((np_,), jnp.float32),
            pltpu.VMEM((BATCH // SUB, SUB), jnp.int32),
            pltpu.VMEM((BATCH,), jnp.float32),
            pltpu.VMEM((np_ // NTILES,), jnp.float32),
        ],
    )(col, ew)


GRP = 256
NSLOT = 4


def _sc_edge_body(nchunks, np_, nsub_pt, zn,
                  y, row, col, ew, out,
                  acc, rowg, colg, ewg, rows, zb, zcol,
                  lsem0, lsem1, gsem0, gsem1, ssem0, ssem1):
    cid = lax.axis_index("c")
    sid = lax.axis_index("s")
    nprt = np_ // NTILES
    spg = GRP // SUB
    ng = (nsub_pt * SUB) // GRP
    wid = cid * NTILES + sid
    tile_e = wid * (nsub_pt * SUB)
    lsem = (lsem0, lsem1)
    gsem = (gsem0, gsem1)
    ssem = (ssem0, ssem1)

    @pl.loop(0, zn)
    def _(i):
        zb[i] = jnp.zeros((LANES,), jnp.float32)

    @pl.loop(0, spg)
    def _(i):
        for j in range(SUB // LANES):
            zcol[i, pl.ds(j * LANES, LANES)] = jnp.zeros((LANES,), jnp.int32)

    def fire_linear(p, s, g):
        pltpu.async_copy(row.at[pl.ds(tile_e + g * GRP, GRP)],
                         rowg.at[p], lsem[p])
        pltpu.async_copy(ew.at[pl.ds(tile_e + g * GRP, GRP)],
                         ewg.at[p], lsem[p])
        for j in range(spg):
            pltpu.async_copy(col.at[pl.ds(tile_e + g * GRP + j * SUB, SUB)],
                             colg.at[s].at[j], lsem[p])

    def wait_linear(p, s, g):
        pltpu.make_async_copy(row.at[pl.ds(tile_e + g * GRP, GRP)],
                              rowg.at[p], lsem[p]).wait()
        pltpu.make_async_copy(ew.at[pl.ds(tile_e + g * GRP, GRP)],
                              ewg.at[p], lsem[p]).wait()
        for j in range(spg):
            pltpu.make_async_copy(
                col.at[pl.ds(tile_e + g * GRP + j * SUB, SUB)],
                colg.at[s].at[j], lsem[p]).wait()

    def fire_gather(c, p):
        for j in range(spg):
            pltpu.async_copy(
                y.at[c].at[rowg.at[p].at[pl.ds(j * SUB, SUB)]],
                rows.at[p].at[pl.ds(j * SUB, SUB)], gsem[p])

    def wait_gather(c, p):
        for j in range(spg):
            pltpu.make_async_copy(
                y.at[c].at[rowg.at[p].at[pl.ds(j * SUB, SUB)]],
                rows.at[p].at[pl.ds(j * SUB, SUB)], gsem[p]).wait()

    def fire_scatter(p, s):
        for j in range(spg):
            pltpu.async_copy(rows.at[p].at[pl.ds(j * SUB, SUB)],
                             acc.at[colg.at[s].at[j]], ssem[p], add=True)

    def wait_scatter(p, s):
        for j in range(spg):
            pltpu.make_async_copy(rows.at[p].at[pl.ds(j * SUB, SUB)],
                                  acc.at[colg.at[s].at[j]],
                                  ssem[p]).wait()

    def fire_prime(p):
        for j in range(spg):
            pltpu.async_copy(rows.at[p].at[pl.ds(j * SUB, SUB)],
                             acc.at[zcol.at[j]], ssem[p], add=True)

    def wait_prime(p):
        for j in range(spg):
            pltpu.make_async_copy(rows.at[p].at[pl.ds(j * SUB, SUB)],
                                  acc.at[zcol.at[j]], ssem[p]).wait()

    def scale(p):
        @pl.loop(0, GRP // LANES)
        def _(g):
            e0 = g * LANES
            ewv = ewg[p, pl.ds(e0, LANES)]
            for j in range(LANES):
                w16 = jnp.full((LANES,), ewv[j], jnp.float32)
                rows[p, e0 + j] = rows[p, e0 + j] * w16

    for c in range(nchunks):
        @pl.loop(0, nprt // zn)
        def _(i):
            pltpu.sync_copy(zb, acc.at[pl.ds(sid * nprt + i * zn, zn)])

        plsc.subcore_barrier()

        @pl.loop(0, GRP)
        def _(i):
            rows[1, i] = jnp.zeros((LANES,), jnp.float32)

        fire_prime(1)
        pltpu.sync_copy(row.at[pl.ds(tile_e, GRP)], rowg.at[0])
        pltpu.sync_copy(ew.at[pl.ds(tile_e, GRP)], ewg.at[0])
        for j in range(spg):
            pltpu.sync_copy(col.at[pl.ds(tile_e + j * SUB, SUB)],
                            colg.at[0].at[j])
        fire_gather(c, 0)
        fire_linear(1, 1, 1)

        @pl.loop(0, ng // NSLOT)
        def _(i):
            for u in range(NSLOT):
                g = i * NSLOT + u
                p = u % 2
                q = 1 - p
                wait_gather(c, p)
                wait_linear(q, (u + 1) % NSLOT, g + 1)
                wait_scatter(q, (u + 3) % NSLOT)
                fire_gather(c, q)
                scale(p)
                fire_scatter(p, u % NSLOT)
                fire_linear(p, (u + 2) % NSLOT, g + 2)

        wait_gather(c, 0)
        wait_linear(1, (ng + 1) % NSLOT, ng + 1)
        wait_scatter(1, (ng - 1) % NSLOT)

        plsc.subcore_barrier()
        pltpu.sync_copy(acc.at[pl.ds(sid * nprt, nprt)],
                        out.at[cid].at[c].at[pl.ds(sid * nprt, nprt)])
        plsc.subcore_barrier()


def _sc_edge(y, row, col, ew, np_, ep):
    nchunks = y.shape[0]
    nsub_pt = ep // (NCORES * NTILES * SUB)
    zn = 98
    mesh = plsc.VectorSubcoreMesh(core_axis_name="c", subcore_axis_name="s")
    return pl.kernel(
        functools.partial(_sc_edge_body, nchunks, np_, nsub_pt, zn),
        out_type=jax.ShapeDtypeStruct((NCORES, nchunks, np_, LANES),
                                      jnp.float32),
        mesh=mesh,
        compiler_params=pltpu.CompilerParams(use_tc_tiling_on_sc=False),
        scratch_types=[
            pltpu.VMEM_SHARED((np_, LANES), jnp.float32),
            pltpu.VMEM((2, GRP), jnp.int32),
            pltpu.VMEM((NSLOT, GRP // SUB, SUB), jnp.int32),
            pltpu.VMEM((2, GRP), jnp.float32),
            pltpu.VMEM((2, GRP, LANES), jnp.float32),
            pltpu.VMEM((zn, LANES), jnp.float32),
            pltpu.VMEM((GRP // SUB, SUB), jnp.int32),
            pltpu.SemaphoreType.DMA,
            pltpu.SemaphoreType.DMA,
            pltpu.SemaphoreType.DMA,
            pltpu.SemaphoreType.DMA,
            pltpu.SemaphoreType.DMA,
            pltpu.SemaphoreType.DMA,
        ],
    )(y, row, col, ew)


def _tc_prep(degp, x, w1p, np_):
    nblk = np_ // BN
    fin = x.shape[1]

    def body(degp_ref, x_ref, w_ref, dinv_ref, y_ref):
        deg = degp_ref[0] + degp_ref[1] + 1.0
        dinv = jnp.where(deg > 0, lax.rsqrt(deg), 0.0)
        dinv_ref[...] = dinv[:, None]
        xw = jnp.dot(x_ref[...], w_ref[...],
                     preferred_element_type=jnp.float32)
        y_ref[0] = xw * dinv[:, None]

    return pl.pallas_call(
        body,
        grid=(nblk,),
        in_specs=[
            pl.BlockSpec((NCORES, BN), lambda i: (0, i)),
            pl.BlockSpec((BN, fin), lambda i: (i, 0)),
            pl.BlockSpec(w1p.shape, lambda i: (0, 0)),
        ],
        out_specs=[
            pl.BlockSpec((BN, 1), lambda i: (i, 0)),
            pl.BlockSpec((1, BN, LANES), lambda i: (0, i, 0)),
        ],
        out_shape=[
            jax.ShapeDtypeStruct((np_, 1), jnp.float32),
            jax.ShapeDtypeStruct((1, np_, LANES), jnp.float32),
        ],
    )(degp, x, w1p)


def _tc_boundary(S, y, dinv, bp, wn, np_):
    kin = y.shape[0]
    kout = wn.shape[1] // LANES
    nblk = np_ // BN

    def body(s_ref, y_ref, d_ref, b_ref, w_ref, o_ref):
        tot = s_ref[0] + s_ref[1] + y_ref[...]
        z = jnp.concatenate([tot[c] for c in range(kin)], axis=1)
        dv = d_ref[...]
        h = _leaky(z * dv + b_ref[...])
        yn = jnp.dot(h, w_ref[...], preferred_element_type=jnp.float32) * dv
        for c in range(kout):
            o_ref[c] = yn[:, c * LANES:(c + 1) * LANES]

    return pl.pallas_call(
        body,
        grid=(nblk,),
        in_specs=[
            pl.BlockSpec((NCORES, kin, BN, LANES), lambda i: (0, 0, i, 0)),
            pl.BlockSpec((kin, BN, LANES), lambda i: (0, i, 0)),
            pl.BlockSpec((BN, 1), lambda i: (i, 0)),
            pl.BlockSpec(bp.shape, lambda i: (0, 0)),
            pl.BlockSpec(wn.shape, lambda i: (0, 0)),
        ],
        out_specs=pl.BlockSpec((kout, BN, LANES), lambda i: (0, i, 0)),
        out_shape=jax.ShapeDtypeStruct((kout, np_, LANES), jnp.float32),
    )(S, y, dinv, bp, wn)


def _tc_final(S, y, dinv, bp, bat3, wf1, cf1, wf2, cf2, wf3, cf3,
              np_, ngraphs):
    kin = y.shape[0]
    w = kin * LANES
    nblk = np_ // BN

    def body(s_ref, y_ref, d_ref, b_ref, bat_ref,
             w1_ref, c1_ref, w2_ref, c2_ref, w3_ref, c3_ref, o_ref, pacc):
        i = pl.program_id(0)

        @pl.when(i == 0)
        def _():
            pacc[...] = jnp.zeros_like(pacc)

        tot = s_ref[0] + s_ref[1] + y_ref[...]
        z = jnp.concatenate([tot[c] for c in range(kin)], axis=1)
        h = _leaky(z * d_ref[...] + b_ref[...])
        bi = bat_ref[0, 0, :]
        gid = lax.broadcasted_iota(jnp.int32, (ngraphs, BN), 0)
        onehot = (gid == bi[None, :]).astype(jnp.float32)
        pacc[...] += jnp.dot(onehot, h, preferred_element_type=jnp.float32)

        @pl.when(i == nblk - 1)
        def _():
            p = pacc[...]
            x1 = _leaky(jnp.dot(p, w1_ref[...],
                                preferred_element_type=jnp.float32)
                        + c1_ref[...])
            x2 = _leaky(jnp.dot(x1, w2_ref[...],
                                preferred_element_type=jnp.float32)
                        + c2_ref[...])
            o_ref[...] = jnp.dot(x2, w3_ref[...],
                                 preferred_element_type=jnp.float32) \
                + c3_ref[...]

    return pl.pallas_call(
        body,
        grid=(nblk,),
        in_specs=[
            pl.BlockSpec((NCORES, kin, BN, LANES), lambda i: (0, 0, i, 0)),
            pl.BlockSpec((kin, BN, LANES), lambda i: (0, i, 0)),
            pl.BlockSpec((BN, 1), lambda i: (i, 0)),
            pl.BlockSpec(bp.shape, lambda i: (0, 0)),
            pl.BlockSpec((1, 1, BN), lambda i: (i, 0, 0)),
            pl.BlockSpec(wf1.shape, lambda i: (0, 0)),
            pl.BlockSpec(cf1.shape, lambda i: (0, 0)),
            pl.BlockSpec(wf2.shape, lambda i: (0, 0)),
            pl.BlockSpec(cf2.shape, lambda i: (0, 0)),
            pl.BlockSpec(wf3.shape, lambda i: (0, 0)),
            pl.BlockSpec(cf3.shape, lambda i: (0, 0)),
        ],
        out_specs=pl.BlockSpec((ngraphs, wf3.shape[1]), lambda i: (0, 0)),
        out_shape=jax.ShapeDtypeStruct((ngraphs, wf3.shape[1]), jnp.float32),
        scratch_shapes=[pltpu.VMEM((ngraphs, w), jnp.float32)],
    )(S, y, dinv, bp, bat3, wf1, cf1, wf2, cf2, wf3, cf3)


def _padw(w, b):
    fin, fout = w.shape
    fin_p = -(-fin // LANES) * LANES if fin > 4 else fin
    fout_p = -(-fout // LANES) * LANES
    wp = jnp.pad(w, ((0, fin_p - fin), (0, fout_p - fout)))
    bp = jnp.pad(b, (0, fout_p - fout)).reshape(1, fout_p)
    return wp, bp


def kernel(node_features, edge_index, edge_weight, batch_index,
           W1, b1, W2, b2, W3, b3, Wf1, bf1, Wf2, bf2, Wf3, bf3):
    n0 = node_features.shape[0]
    e0 = edge_index.shape[1]
    ngraphs = 64

    np_ = -(-n0 // BN) * BN
    epb = NCORES * NTILES * BATCH
    ep = -(-e0 // epb) * epb

    x = jnp.pad(node_features, ((0, np_ - n0), (0, 0)))
    row = jnp.pad(edge_index[0], (0, ep + BATCH - e0))
    col = jnp.pad(edge_index[1], (0, ep + BATCH - e0))
    ew = jnp.pad(edge_weight, (0, ep + BATCH - e0))
    bat = jnp.pad(batch_index, (0, np_ - n0), constant_values=ngraphs)
    bat3 = bat.reshape(np_ // BN, 1, BN)

    w1p, b1p = _padw(W1, b1)
    w2p, b2p = _padw(W2, b2)
    w3p, b3p = _padw(W3, b3)
    cf1 = bf1.reshape(1, -1)
    cf2 = bf2.reshape(1, -1)
    cf3 = bf3.reshape(1, -1)

    degp = _sc_deg(col, ew, np_, ep)
    dinv, y1 = _tc_prep(degp, x, w1p, np_)
    s1 = _sc_edge(y1, row, col, ew, np_, ep)
    y2 = _tc_boundary(s1, y1, dinv, b1p, w2p, np_)
    s2 = _sc_edge(y2, row, col, ew, np_, ep)
    y3 = _tc_boundary(s2, y2, dinv, b2p, w3p, np_)
    s3 = _sc_edge(y3, row, col, ew, np_, ep)
    return _tc_final(s3, y3, dinv, b3p, bat3,
                     Wf1, cf1, Wf2, cf2, Wf3, cf3, np_, ngraphs)


# speedup vs baseline: 1.0660x; 1.0660x over previous
import functools

import jax
import jax.numpy as jnp
from jax import lax
from jax.experimental import pallas as pl
from jax.experimental.pallas import tpu as pltpu
from jax.experimental.pallas import tpu_sc as plsc

LANES = 16
SUB = 128
BATCH = 512
NCORES = 2
NTILES = 16
BN = 3584


def _leaky(x):
    return jnp.where(x >= 0, x, 0.1 * x)


def _sc_deg_body(np_, nsub_pt, col2, ew, out, dacc, colb, ewb, zb):
    cid = lax.axis_index("c")
    sid = lax.axis_index("s")
    npt = np_ // NTILES
    sub_pb = BATCH // SUB

    @pl.loop(0, npt // LANES)
    def _(i):
        zb[pl.ds(i * LANES, LANES)] = jnp.zeros((LANES,), jnp.float32)

    pltpu.sync_copy(zb, dacc.at[pl.ds(sid * npt, npt)])
    plsc.subcore_barrier()

    wid = cid * NTILES + sid
    nb = nsub_pt // sub_pb

    @pl.loop(0, nb)
    def _(b):
        eb = wid * (nsub_pt * SUB) + b * BATCH
        sb = wid * nsub_pt + b * sub_pb
        pltpu.sync_copy(col2.at[pl.ds(sb, sub_pb)], colb)
        pltpu.sync_copy(ew.at[pl.ds(eb, BATCH)], ewb)
        for j in range(sub_pb):
            pltpu.sync_copy(ewb.at[pl.ds(j * SUB, SUB)],
                            dacc.at[colb.at[j]], add=True)

    plsc.subcore_barrier()
    pltpu.sync_copy(dacc.at[pl.ds(sid * npt, npt)],
                    out.at[cid].at[pl.ds(sid * npt, npt)])


def _sc_deg(col2, ew, np_, ep):
    nsub_pt = ep // (NCORES * NTILES * SUB)
    mesh = plsc.VectorSubcoreMesh(core_axis_name="c", subcore_axis_name="s")
    return pl.kernel(
        functools.partial(_sc_deg_body, np_, nsub_pt),
        out_type=jax.ShapeDtypeStruct((NCORES, np_), jnp.float32),
        mesh=mesh,
        compiler_params=pltpu.CompilerParams(use_tc_tiling_on_sc=False),
        scratch_types=[
            pltpu.VMEM_SHARED((np_,), jnp.float32),
            pltpu.VMEM((BATCH // SUB, SUB), jnp.int32),
            pltpu.VMEM((BATCH,), jnp.float32),
            pltpu.VMEM((np_ // NTILES,), jnp.float32),
        ],
    )(col2, ew)


GRP = 256
NSLOT = 4


def _sc_edge_body(nchunks, np_, nsub_pt, zn,
                  y, row, col2, ew, out,
                  acc, rowg, colg, ewg, rows, zb, zcol,
                  lsem0, lsem1, gsem0, gsem1, ssem0, ssem1):
    cid = lax.axis_index("c")
    sid = lax.axis_index("s")
    nprt = np_ // NTILES
    spg = GRP // SUB
    ng = (nsub_pt * SUB) // GRP
    wid = cid * NTILES + sid
    tile_e = wid * (nsub_pt * SUB)
    tile_s = wid * nsub_pt
    lsem = (lsem0, lsem1)
    gsem = (gsem0, gsem1)
    ssem = (ssem0, ssem1)

    @pl.loop(0, zn)
    def _(i):
        zb[i] = jnp.zeros((LANES,), jnp.float32)

    @pl.loop(0, spg)
    def _(i):
        for j in range(SUB // LANES):
            zcol[i, pl.ds(j * LANES, LANES)] = jnp.zeros((LANES,), jnp.int32)

    def fire_linear(p, s, g):
        pltpu.async_copy(row.at[pl.ds(tile_e + g * GRP, GRP)],
                         rowg.at[p], lsem[p])
        pltpu.async_copy(ew.at[pl.ds(tile_e + g * GRP, GRP)],
                         ewg.at[p], lsem[p])
        pltpu.async_copy(col2.at[pl.ds(tile_s + g * spg, spg)],
                         colg.at[s], lsem[p])

    def wait_linear(p, s, g):
        pltpu.make_async_copy(row.at[pl.ds(tile_e + g * GRP, GRP)],
                              rowg.at[p], lsem[p]).wait()
        pltpu.make_async_copy(ew.at[pl.ds(tile_e + g * GRP, GRP)],
                              ewg.at[p], lsem[p]).wait()
        pltpu.make_async_copy(col2.at[pl.ds(tile_s + g * spg, spg)],
                              colg.at[s], lsem[p]).wait()

    def fire_gather(c, p):
        for j in range(spg):
            pltpu.async_copy(
                y.at[c].at[rowg.at[p].at[pl.ds(j * SUB, SUB)]],
                rows.at[p].at[pl.ds(j * SUB, SUB)], gsem[p])

    def wait_gather(c, p):
        for j in range(spg):
            pltpu.make_async_copy(
                y.at[c].at[rowg.at[p].at[pl.ds(j * SUB, SUB)]],
                rows.at[p].at[pl.ds(j * SUB, SUB)], gsem[p]).wait()

    def fire_scatter(p, s):
        for j in range(spg):
            pltpu.async_copy(rows.at[p].at[pl.ds(j * SUB, SUB)],
                             acc.at[colg.at[s].at[j]], ssem[p], add=True)

    def wait_scatter(p, s):
        for j in range(spg):
            pltpu.make_async_copy(rows.at[p].at[pl.ds(j * SUB, SUB)],
                                  acc.at[colg.at[s].at[j]],
                                  ssem[p]).wait()

    def fire_prime(p):
        for j in range(spg):
            pltpu.async_copy(rows.at[p].at[pl.ds(j * SUB, SUB)],
                             acc.at[zcol.at[j]], ssem[p], add=True)

    def wait_prime(p):
        for j in range(spg):
            pltpu.make_async_copy(rows.at[p].at[pl.ds(j * SUB, SUB)],
                                  acc.at[zcol.at[j]], ssem[p]).wait()

    def scale(p):
        @pl.loop(0, GRP // LANES)
        def _(g):
            e0 = g * LANES
            ewv = ewg[p, pl.ds(e0, LANES)]
            for j in range(LANES):
                w16 = jnp.full((LANES,), ewv[j], jnp.float32)
                rows[p, e0 + j] = rows[p, e0 + j] * w16

    for c in range(nchunks):
        @pl.loop(0, nprt // zn)
        def _(i):
            pltpu.sync_copy(zb, acc.at[pl.ds(sid * nprt + i * zn, zn)])

        plsc.subcore_barrier()

        @pl.loop(0, GRP)
        def _(i):
            rows[1, i] = jnp.zeros((LANES,), jnp.float32)

        fire_prime(1)
        pltpu.sync_copy(row.at[pl.ds(tile_e, GRP)], rowg.at[0])
        pltpu.sync_copy(ew.at[pl.ds(tile_e, GRP)], ewg.at[0])
        pltpu.sync_copy(col2.at[pl.ds(tile_s, spg)], colg.at[0])
        fire_gather(c, 0)
        fire_linear(1, 1, 1)

        @pl.loop(0, ng // NSLOT)
        def _(i):
            for u in range(NSLOT):
                g = i * NSLOT + u
                p = u % 2
                q = 1 - p
                wait_gather(c, p)
                wait_linear(q, (u + 1) % NSLOT, g + 1)
                wait_scatter(q, (u + 3) % NSLOT)
                fire_gather(c, q)
                scale(p)
                fire_scatter(p, u % NSLOT)
                fire_linear(p, (u + 2) % NSLOT, g + 2)

        wait_gather(c, 0)
        wait_linear(1, (ng + 1) % NSLOT, ng + 1)
        wait_scatter(1, (ng - 1) % NSLOT)

        plsc.subcore_barrier()
        pltpu.sync_copy(acc.at[pl.ds(sid * nprt, nprt)],
                        out.at[cid].at[c].at[pl.ds(sid * nprt, nprt)])
        plsc.subcore_barrier()


def _sc_edge(y, row, col2, ew, np_, ep):
    nchunks = y.shape[0]
    nsub_pt = ep // (NCORES * NTILES * SUB)
    zn = 98
    mesh = plsc.VectorSubcoreMesh(core_axis_name="c", subcore_axis_name="s")
    return pl.kernel(
        functools.partial(_sc_edge_body, nchunks, np_, nsub_pt, zn),
        out_type=jax.ShapeDtypeStruct((NCORES, nchunks, np_, LANES),
                                      jnp.float32),
        mesh=mesh,
        compiler_params=pltpu.CompilerParams(use_tc_tiling_on_sc=False),
        scratch_types=[
            pltpu.VMEM_SHARED((np_, LANES), jnp.float32),
            pltpu.VMEM((2, GRP), jnp.int32),
            pltpu.VMEM((NSLOT, GRP // SUB, SUB), jnp.int32),
            pltpu.VMEM((2, GRP), jnp.float32),
            pltpu.VMEM((2, GRP, LANES), jnp.float32),
            pltpu.VMEM((zn, LANES), jnp.float32),
            pltpu.VMEM((GRP // SUB, SUB), jnp.int32),
            pltpu.SemaphoreType.DMA,
            pltpu.SemaphoreType.DMA,
            pltpu.SemaphoreType.DMA,
            pltpu.SemaphoreType.DMA,
            pltpu.SemaphoreType.DMA,
            pltpu.SemaphoreType.DMA,
        ],
    )(y, row, col2, ew)


def _tc_prep(degp, x, w1p, np_):
    nblk = np_ // BN
    fin = x.shape[1]

    def body(degp_ref, x_ref, w_ref, dinv_ref, y_ref):
        deg = degp_ref[0] + degp_ref[1] + 1.0
        dinv = jnp.where(deg > 0, lax.rsqrt(deg), 0.0)
        dinv_ref[...] = dinv[:, None]
        xw = jnp.dot(x_ref[...], w_ref[...],
                     preferred_element_type=jnp.float32)
        y_ref[0] = xw * dinv[:, None]

    return pl.pallas_call(
        body,
        grid=(nblk,),
        in_specs=[
            pl.BlockSpec((NCORES, BN), lambda i: (0, i)),
            pl.BlockSpec((BN, fin), lambda i: (i, 0)),
            pl.BlockSpec(w1p.shape, lambda i: (0, 0)),
        ],
        out_specs=[
            pl.BlockSpec((BN, 1), lambda i: (i, 0)),
            pl.BlockSpec((1, BN, LANES), lambda i: (0, i, 0)),
        ],
        out_shape=[
            jax.ShapeDtypeStruct((np_, 1), jnp.float32),
            jax.ShapeDtypeStruct((1, np_, LANES), jnp.float32),
        ],
    )(degp, x, w1p)


def _tc_boundary(S, y, dinv, bp, wn, np_):
    kin = y.shape[0]
    kout = wn.shape[1] // LANES
    nblk = np_ // BN

    def body(s_ref, y_ref, d_ref, b_ref, w_ref, o_ref):
        tot = s_ref[0] + s_ref[1] + y_ref[...]
        z = jnp.concatenate([tot[c] for c in range(kin)], axis=1)
        dv = d_ref[...]
        h = _leaky(z * dv + b_ref[...])
        yn = jnp.dot(h, w_ref[...], preferred_element_type=jnp.float32) * dv
        for c in range(kout):
            o_ref[c] = yn[:, c * LANES:(c + 1) * LANES]

    return pl.pallas_call(
        body,
        grid=(nblk,),
        in_specs=[
            pl.BlockSpec((NCORES, kin, BN, LANES), lambda i: (0, 0, i, 0)),
            pl.BlockSpec((kin, BN, LANES), lambda i: (0, i, 0)),
            pl.BlockSpec((BN, 1), lambda i: (i, 0)),
            pl.BlockSpec(bp.shape, lambda i: (0, 0)),
            pl.BlockSpec(wn.shape, lambda i: (0, 0)),
        ],
        out_specs=pl.BlockSpec((kout, BN, LANES), lambda i: (0, i, 0)),
        out_shape=jax.ShapeDtypeStruct((kout, np_, LANES), jnp.float32),
    )(S, y, dinv, bp, wn)


def _tc_final(S, y, dinv, bp, bat3, wf1, cf1, wf2, cf2, wf3, cf3,
              np_, ngraphs):
    kin = y.shape[0]
    w = kin * LANES
    nblk = np_ // BN

    def body(s_ref, y_ref, d_ref, b_ref, bat_ref,
             w1_ref, c1_ref, w2_ref, c2_ref, w3_ref, c3_ref, o_ref, pacc):
        i = pl.program_id(0)

        @pl.when(i == 0)
        def _():
            pacc[...] = jnp.zeros_like(pacc)

        tot = s_ref[0] + s_ref[1] + y_ref[...]
        z = jnp.concatenate([tot[c] for c in range(kin)], axis=1)
        h = _leaky(z * d_ref[...] + b_ref[...])
        bi = bat_ref[0, 0, :]
        gid = lax.broadcasted_iota(jnp.int32, (ngraphs, BN), 0)
        onehot = (gid == bi[None, :]).astype(jnp.float32)
        pacc[...] += jnp.dot(onehot, h, preferred_element_type=jnp.float32)

        @pl.when(i == nblk - 1)
        def _():
            p = pacc[...]
            x1 = _leaky(jnp.dot(p, w1_ref[...],
                                preferred_element_type=jnp.float32)
                        + c1_ref[...])
            x2 = _leaky(jnp.dot(x1, w2_ref[...],
                                preferred_element_type=jnp.float32)
                        + c2_ref[...])
            o_ref[...] = jnp.dot(x2, w3_ref[...],
                                 preferred_element_type=jnp.float32) \
                + c3_ref[...]

    return pl.pallas_call(
        body,
        grid=(nblk,),
        in_specs=[
            pl.BlockSpec((NCORES, kin, BN, LANES), lambda i: (0, 0, i, 0)),
            pl.BlockSpec((kin, BN, LANES), lambda i: (0, i, 0)),
            pl.BlockSpec((BN, 1), lambda i: (i, 0)),
            pl.BlockSpec(bp.shape, lambda i: (0, 0)),
            pl.BlockSpec((1, 1, BN), lambda i: (i, 0, 0)),
            pl.BlockSpec(wf1.shape, lambda i: (0, 0)),
            pl.BlockSpec(cf1.shape, lambda i: (0, 0)),
            pl.BlockSpec(wf2.shape, lambda i: (0, 0)),
            pl.BlockSpec(cf2.shape, lambda i: (0, 0)),
            pl.BlockSpec(wf3.shape, lambda i: (0, 0)),
            pl.BlockSpec(cf3.shape, lambda i: (0, 0)),
        ],
        out_specs=pl.BlockSpec((ngraphs, wf3.shape[1]), lambda i: (0, 0)),
        out_shape=jax.ShapeDtypeStruct((ngraphs, wf3.shape[1]), jnp.float32),
        scratch_shapes=[pltpu.VMEM((ngraphs, w), jnp.float32)],
    )(S, y, dinv, bp, bat3, wf1, cf1, wf2, cf2, wf3, cf3)


def _padw(w, b):
    fin, fout = w.shape
    fin_p = -(-fin // LANES) * LANES if fin > 4 else fin
    fout_p = -(-fout // LANES) * LANES
    wp = jnp.pad(w, ((0, fin_p - fin), (0, fout_p - fout)))
    bp = jnp.pad(b, (0, fout_p - fout)).reshape(1, fout_p)
    return wp, bp


def kernel(node_features, edge_index, edge_weight, batch_index,
           W1, b1, W2, b2, W3, b3, Wf1, bf1, Wf2, bf2, Wf3, bf3):
    n0 = node_features.shape[0]
    e0 = edge_index.shape[1]
    ngraphs = 64

    np_ = -(-n0 // BN) * BN
    epb = NCORES * NTILES * BATCH
    ep = -(-e0 // epb) * epb

    x = jnp.pad(node_features, ((0, np_ - n0), (0, 0)))
    row = jnp.pad(edge_index[0], (0, ep + BATCH - e0))
    col = jnp.pad(edge_index[1], (0, ep + BATCH - e0))
    col2 = col.reshape((ep + BATCH) // SUB, SUB)
    ew = jnp.pad(edge_weight, (0, ep + BATCH - e0))
    bat = jnp.pad(batch_index, (0, np_ - n0), constant_values=ngraphs)
    bat3 = bat.reshape(np_ // BN, 1, BN)

    w1p, b1p = _padw(W1, b1)
    w2p, b2p = _padw(W2, b2)
    w3p, b3p = _padw(W3, b3)
    cf1 = bf1.reshape(1, -1)
    cf2 = bf2.reshape(1, -1)
    cf3 = bf3.reshape(1, -1)

    degp = _sc_deg(col2, ew, np_, ep)
    dinv, y1 = _tc_prep(degp, x, w1p, np_)
    s1 = _sc_edge(y1, row, col2, ew, np_, ep)
    y2 = _tc_boundary(s1, y1, dinv, b1p, w2p, np_)
    s2 = _sc_edge(y2, row, col2, ew, np_, ep)
    y3 = _tc_boundary(s2, y2, dinv, b2p, w3p, np_)
    s3 = _sc_edge(y3, row, col2, ew, np_, ep)
    return _tc_final(s3, y3, dinv, b3p, bat3,
                     Wf1, cf1, Wf2, cf2, Wf3, cf3, np_, ngraphs)


# speedup vs baseline: 1.2507x; 1.1732x over previous
import functools

import jax
import jax.numpy as jnp
from jax import lax
from jax.experimental import pallas as pl
from jax.experimental.pallas import tpu as pltpu
from jax.experimental.pallas import tpu_sc as plsc

LANES = 16
SUB = 128
BATCH = 512
NCORES = 2
NTILES = 16
BN = 3584


def _leaky(x):
    return jnp.where(x >= 0, x, 0.1 * x)


def _sc_deg_body(np_, nsub_pt, col2, ew, out, dacc, colb, ewb, zb):
    cid = lax.axis_index("c")
    sid = lax.axis_index("s")
    npt = np_ // NTILES
    sub_pb = BATCH // SUB

    @pl.loop(0, npt // LANES)
    def _(i):
        zb[pl.ds(i * LANES, LANES)] = jnp.zeros((LANES,), jnp.float32)

    pltpu.sync_copy(zb, dacc.at[pl.ds(sid * npt, npt)])
    plsc.subcore_barrier()

    wid = cid * NTILES + sid
    nb = nsub_pt // sub_pb

    @pl.loop(0, nb)
    def _(b):
        eb = wid * (nsub_pt * SUB) + b * BATCH
        sb = wid * nsub_pt + b * sub_pb
        pltpu.sync_copy(col2.at[pl.ds(sb, sub_pb)], colb)
        pltpu.sync_copy(ew.at[pl.ds(eb, BATCH)], ewb)
        for j in range(sub_pb):
            pltpu.sync_copy(ewb.at[pl.ds(j * SUB, SUB)],
                            dacc.at[colb.at[j]], add=True)

    plsc.subcore_barrier()
    pltpu.sync_copy(dacc.at[pl.ds(sid * npt, npt)],
                    out.at[cid].at[pl.ds(sid * npt, npt)])


def _sc_deg(col2, ew, np_, ep):
    nsub_pt = ep // (NCORES * NTILES * SUB)
    mesh = plsc.VectorSubcoreMesh(core_axis_name="c", subcore_axis_name="s")
    return pl.kernel(
        functools.partial(_sc_deg_body, np_, nsub_pt),
        out_type=jax.ShapeDtypeStruct((NCORES, np_), jnp.float32),
        mesh=mesh,
        compiler_params=pltpu.CompilerParams(use_tc_tiling_on_sc=False),
        scratch_types=[
            pltpu.VMEM_SHARED((np_,), jnp.float32),
            pltpu.VMEM((BATCH // SUB, SUB), jnp.int32),
            pltpu.VMEM((BATCH,), jnp.float32),
            pltpu.VMEM((np_ // NTILES,), jnp.float32),
        ],
    )(col2, ew)


GRP = 512
NSLOT = 4


def _sc_edge_body(nchunks, np_, nsub_pt, zn,
                  y, row, col2, ew, out,
                  acc, rowg, colg, ewg, rows, zb, zcol,
                  lsem0, lsem1, gsem0, gsem1, ssem0, ssem1):
    cid = lax.axis_index("c")
    sid = lax.axis_index("s")
    nprt = np_ // NTILES
    spg = GRP // SUB
    ng = (nsub_pt * SUB) // GRP
    wid = cid * NTILES + sid
    tile_e = wid * (nsub_pt * SUB)
    tile_s = wid * nsub_pt
    lsem = (lsem0, lsem1)
    gsem = (gsem0, gsem1)
    ssem = (ssem0, ssem1)

    @pl.loop(0, zn)
    def _(i):
        zb[i] = jnp.zeros((LANES,), jnp.float32)

    @pl.loop(0, spg)
    def _(i):
        for j in range(SUB // LANES):
            zcol[i, pl.ds(j * LANES, LANES)] = jnp.zeros((LANES,), jnp.int32)

    def fire_linear(p, s, g):
        pltpu.async_copy(row.at[pl.ds(tile_e + g * GRP, GRP)],
                         rowg.at[p], lsem[p])
        pltpu.async_copy(ew.at[pl.ds(tile_e + g * GRP, GRP)],
                         ewg.at[p], lsem[p])
        pltpu.async_copy(col2.at[pl.ds(tile_s + g * spg, spg)],
                         colg.at[s], lsem[p])

    def wait_linear(p, s, g):
        pltpu.make_async_copy(row.at[pl.ds(tile_e + g * GRP, GRP)],
                              rowg.at[p], lsem[p]).wait()
        pltpu.make_async_copy(ew.at[pl.ds(tile_e + g * GRP, GRP)],
                              ewg.at[p], lsem[p]).wait()
        pltpu.make_async_copy(col2.at[pl.ds(tile_s + g * spg, spg)],
                              colg.at[s], lsem[p]).wait()

    def fire_gather(c, p):
        for j in range(spg):
            pltpu.async_copy(
                y.at[c].at[rowg.at[p].at[pl.ds(j * SUB, SUB)]],
                rows.at[p].at[pl.ds(j * SUB, SUB)], gsem[p])

    def wait_gather(c, p):
        for j in range(spg):
            pltpu.make_async_copy(
                y.at[c].at[rowg.at[p].at[pl.ds(j * SUB, SUB)]],
                rows.at[p].at[pl.ds(j * SUB, SUB)], gsem[p]).wait()

    def fire_scatter(p, s):
        for j in range(spg):
            pltpu.async_copy(rows.at[p].at[pl.ds(j * SUB, SUB)],
                             acc.at[colg.at[s].at[j]], ssem[p], add=True)

    def wait_scatter(p, s):
        for j in range(spg):
            pltpu.make_async_copy(rows.at[p].at[pl.ds(j * SUB, SUB)],
                                  acc.at[colg.at[s].at[j]],
                                  ssem[p]).wait()

    def fire_prime(p):
        for j in range(spg):
            pltpu.async_copy(rows.at[p].at[pl.ds(j * SUB, SUB)],
                             acc.at[zcol.at[j]], ssem[p], add=True)

    def wait_prime(p):
        for j in range(spg):
            pltpu.make_async_copy(rows.at[p].at[pl.ds(j * SUB, SUB)],
                                  acc.at[zcol.at[j]], ssem[p]).wait()

    def scale(p):
        @pl.loop(0, GRP // LANES)
        def _(g):
            e0 = g * LANES
            ewv = ewg[p, pl.ds(e0, LANES)]
            for j in range(LANES):
                w16 = jnp.full((LANES,), ewv[j], jnp.float32)
                rows[p, e0 + j] = rows[p, e0 + j] * w16

    for c in range(nchunks):
        @pl.loop(0, nprt // zn)
        def _(i):
            pltpu.sync_copy(zb, acc.at[pl.ds(sid * nprt + i * zn, zn)])

        plsc.subcore_barrier()

        @pl.loop(0, GRP)
        def _(i):
            rows[1, i] = jnp.zeros((LANES,), jnp.float32)

        fire_prime(1)
        pltpu.sync_copy(row.at[pl.ds(tile_e, GRP)], rowg.at[0])
        pltpu.sync_copy(ew.at[pl.ds(tile_e, GRP)], ewg.at[0])
        pltpu.sync_copy(col2.at[pl.ds(tile_s, spg)], colg.at[0])
        fire_gather(c, 0)
        fire_linear(1, 1, 1)

        @pl.loop(0, ng // NSLOT)
        def _(i):
            for u in range(NSLOT):
                g = i * NSLOT + u
                p = u % 2
                q = 1 - p
                wait_gather(c, p)
                wait_linear(q, (u + 1) % NSLOT, g + 1)
                wait_scatter(q, (u + 3) % NSLOT)
                fire_gather(c, q)
                scale(p)
                fire_scatter(p, u % NSLOT)
                fire_linear(p, (u + 2) % NSLOT, g + 2)

        wait_gather(c, 0)
        wait_linear(1, (ng + 1) % NSLOT, ng + 1)
        wait_scatter(1, (ng - 1) % NSLOT)

        plsc.subcore_barrier()
        pltpu.sync_copy(acc.at[pl.ds(sid * nprt, nprt)],
                        out.at[cid].at[c].at[pl.ds(sid * nprt, nprt)])
        plsc.subcore_barrier()


def _sc_edge(y, row, col2, ew, np_, ep):
    nchunks = y.shape[0]
    nsub_pt = ep // (NCORES * NTILES * SUB)
    zn = 98
    mesh = plsc.VectorSubcoreMesh(core_axis_name="c", subcore_axis_name="s")
    return pl.kernel(
        functools.partial(_sc_edge_body, nchunks, np_, nsub_pt, zn),
        out_type=jax.ShapeDtypeStruct((NCORES, nchunks, np_, LANES),
                                      jnp.float32),
        mesh=mesh,
        compiler_params=pltpu.CompilerParams(use_tc_tiling_on_sc=False),
        scratch_types=[
            pltpu.VMEM_SHARED((np_, LANES), jnp.float32),
            pltpu.VMEM((2, GRP), jnp.int32),
            pltpu.VMEM((NSLOT, GRP // SUB, SUB), jnp.int32),
            pltpu.VMEM((2, GRP), jnp.float32),
            pltpu.VMEM((2, GRP, LANES), jnp.float32),
            pltpu.VMEM((zn, LANES), jnp.float32),
            pltpu.VMEM((GRP // SUB, SUB), jnp.int32),
            pltpu.SemaphoreType.DMA,
            pltpu.SemaphoreType.DMA,
            pltpu.SemaphoreType.DMA,
            pltpu.SemaphoreType.DMA,
            pltpu.SemaphoreType.DMA,
            pltpu.SemaphoreType.DMA,
        ],
    )(y, row, col2, ew)


def _tc_prep(degp, x, w1p, np_):
    nblk = np_ // BN
    fin = x.shape[1]

    def body(degp_ref, x_ref, w_ref, dinv_ref, y_ref):
        deg = degp_ref[0] + degp_ref[1] + 1.0
        dinv = jnp.where(deg > 0, lax.rsqrt(deg), 0.0)
        dinv_ref[...] = dinv[:, None]
        xw = jnp.dot(x_ref[...], w_ref[...],
                     preferred_element_type=jnp.float32)
        y_ref[0] = xw * dinv[:, None]

    return pl.pallas_call(
        body,
        grid=(nblk,),
        in_specs=[
            pl.BlockSpec((NCORES, BN), lambda i: (0, i)),
            pl.BlockSpec((BN, fin), lambda i: (i, 0)),
            pl.BlockSpec(w1p.shape, lambda i: (0, 0)),
        ],
        out_specs=[
            pl.BlockSpec((BN, 1), lambda i: (i, 0)),
            pl.BlockSpec((1, BN, LANES), lambda i: (0, i, 0)),
        ],
        out_shape=[
            jax.ShapeDtypeStruct((np_, 1), jnp.float32),
            jax.ShapeDtypeStruct((1, np_, LANES), jnp.float32),
        ],
    )(degp, x, w1p)


def _tc_boundary(S, y, dinv, bp, wn, np_):
    kin = y.shape[0]
    kout = wn.shape[1] // LANES
    nblk = np_ // BN

    def body(s_ref, y_ref, d_ref, b_ref, w_ref, o_ref):
        tot = s_ref[0] + s_ref[1] + y_ref[...]
        z = jnp.concatenate([tot[c] for c in range(kin)], axis=1)
        dv = d_ref[...]
        h = _leaky(z * dv + b_ref[...])
        yn = jnp.dot(h, w_ref[...], preferred_element_type=jnp.float32) * dv
        for c in range(kout):
            o_ref[c] = yn[:, c * LANES:(c + 1) * LANES]

    return pl.pallas_call(
        body,
        grid=(nblk,),
        in_specs=[
            pl.BlockSpec((NCORES, kin, BN, LANES), lambda i: (0, 0, i, 0)),
            pl.BlockSpec((kin, BN, LANES), lambda i: (0, i, 0)),
            pl.BlockSpec((BN, 1), lambda i: (i, 0)),
            pl.BlockSpec(bp.shape, lambda i: (0, 0)),
            pl.BlockSpec(wn.shape, lambda i: (0, 0)),
        ],
        out_specs=pl.BlockSpec((kout, BN, LANES), lambda i: (0, i, 0)),
        out_shape=jax.ShapeDtypeStruct((kout, np_, LANES), jnp.float32),
    )(S, y, dinv, bp, wn)


def _tc_final(S, y, dinv, bp, bat3, wf1, cf1, wf2, cf2, wf3, cf3,
              np_, ngraphs):
    kin = y.shape[0]
    w = kin * LANES
    nblk = np_ // BN

    def body(s_ref, y_ref, d_ref, b_ref, bat_ref,
             w1_ref, c1_ref, w2_ref, c2_ref, w3_ref, c3_ref, o_ref, pacc):
        i = pl.program_id(0)

        @pl.when(i == 0)
        def _():
            pacc[...] = jnp.zeros_like(pacc)

        tot = s_ref[0] + s_ref[1] + y_ref[...]
        z = jnp.concatenate([tot[c] for c in range(kin)], axis=1)
        h = _leaky(z * d_ref[...] + b_ref[...])
        bi = bat_ref[0, 0, :]
        gid = lax.broadcasted_iota(jnp.int32, (ngraphs, BN), 0)
        onehot = (gid == bi[None, :]).astype(jnp.float32)
        pacc[...] += jnp.dot(onehot, h, preferred_element_type=jnp.float32)

        @pl.when(i == nblk - 1)
        def _():
            p = pacc[...]
            x1 = _leaky(jnp.dot(p, w1_ref[...],
                                preferred_element_type=jnp.float32)
                        + c1_ref[...])
            x2 = _leaky(jnp.dot(x1, w2_ref[...],
                                preferred_element_type=jnp.float32)
                        + c2_ref[...])
            o_ref[...] = jnp.dot(x2, w3_ref[...],
                                 preferred_element_type=jnp.float32) \
                + c3_ref[...]

    return pl.pallas_call(
        body,
        grid=(nblk,),
        in_specs=[
            pl.BlockSpec((NCORES, kin, BN, LANES), lambda i: (0, 0, i, 0)),
            pl.BlockSpec((kin, BN, LANES), lambda i: (0, i, 0)),
            pl.BlockSpec((BN, 1), lambda i: (i, 0)),
            pl.BlockSpec(bp.shape, lambda i: (0, 0)),
            pl.BlockSpec((1, 1, BN), lambda i: (i, 0, 0)),
            pl.BlockSpec(wf1.shape, lambda i: (0, 0)),
            pl.BlockSpec(cf1.shape, lambda i: (0, 0)),
            pl.BlockSpec(wf2.shape, lambda i: (0, 0)),
            pl.BlockSpec(cf2.shape, lambda i: (0, 0)),
            pl.BlockSpec(wf3.shape, lambda i: (0, 0)),
            pl.BlockSpec(cf3.shape, lambda i: (0, 0)),
        ],
        out_specs=pl.BlockSpec((ngraphs, wf3.shape[1]), lambda i: (0, 0)),
        out_shape=jax.ShapeDtypeStruct((ngraphs, wf3.shape[1]), jnp.float32),
        scratch_shapes=[pltpu.VMEM((ngraphs, w), jnp.float32)],
    )(S, y, dinv, bp, bat3, wf1, cf1, wf2, cf2, wf3, cf3)


def _padw(w, b):
    fin, fout = w.shape
    fin_p = -(-fin // LANES) * LANES if fin > 4 else fin
    fout_p = -(-fout // LANES) * LANES
    wp = jnp.pad(w, ((0, fin_p - fin), (0, fout_p - fout)))
    bp = jnp.pad(b, (0, fout_p - fout)).reshape(1, fout_p)
    return wp, bp


def kernel(node_features, edge_index, edge_weight, batch_index,
           W1, b1, W2, b2, W3, b3, Wf1, bf1, Wf2, bf2, Wf3, bf3):
    n0 = node_features.shape[0]
    e0 = edge_index.shape[1]
    ngraphs = 64

    np_ = -(-n0 // BN) * BN
    epb = NCORES * NTILES * BATCH
    ep = -(-e0 // epb) * epb

    x = jnp.pad(node_features, ((0, np_ - n0), (0, 0)))
    pe = 2 * GRP
    row = jnp.pad(edge_index[0], (0, ep + pe - e0))
    col = jnp.pad(edge_index[1], (0, ep + pe - e0))
    col2 = col.reshape((ep + pe) // SUB, SUB)
    ew = jnp.pad(edge_weight, (0, ep + pe - e0))
    bat = jnp.pad(batch_index, (0, np_ - n0), constant_values=ngraphs)
    bat3 = bat.reshape(np_ // BN, 1, BN)

    w1p, b1p = _padw(W1, b1)
    w2p, b2p = _padw(W2, b2)
    w3p, b3p = _padw(W3, b3)
    cf1 = bf1.reshape(1, -1)
    cf2 = bf2.reshape(1, -1)
    cf3 = bf3.reshape(1, -1)

    degp = _sc_deg(col2, ew, np_, ep)
    dinv, y1 = _tc_prep(degp, x, w1p, np_)
    s1 = _sc_edge(y1, row, col2, ew, np_, ep)
    y2 = _tc_boundary(s1, y1, dinv, b1p, w2p, np_)
    s2 = _sc_edge(y2, row, col2, ew, np_, ep)
    y3 = _tc_boundary(s2, y2, dinv, b2p, w3p, np_)
    s3 = _sc_edge(y3, row, col2, ew, np_, ep)
    return _tc_final(s3, y3, dinv, b3p, bat3,
                     Wf1, cf1, Wf2, cf2, Wf3, cf3, np_, ngraphs)


# speedup vs baseline: 1.2518x; 1.0009x over previous
import functools

import jax
import jax.numpy as jnp
from jax import lax
from jax.experimental import pallas as pl
from jax.experimental.pallas import tpu as pltpu
from jax.experimental.pallas import tpu_sc as plsc

LANES = 16
SUB = 128
BATCH = 512
NCORES = 2
NTILES = 16
BN = 3584


def _leaky(x):
    return jnp.where(x >= 0, x, 0.1 * x)


def _sc_deg_body(np_, nsub_pt, col2, ew, out, dacc, colb, ewb, zb):
    cid = lax.axis_index("c")
    sid = lax.axis_index("s")
    npt = np_ // NTILES
    sub_pb = BATCH // SUB

    @pl.loop(0, npt // LANES)
    def _(i):
        zb[pl.ds(i * LANES, LANES)] = jnp.zeros((LANES,), jnp.float32)

    pltpu.sync_copy(zb, dacc.at[pl.ds(sid * npt, npt)])
    plsc.subcore_barrier()

    wid = cid * NTILES + sid
    nb = nsub_pt // sub_pb

    @pl.loop(0, nb)
    def _(b):
        eb = wid * (nsub_pt * SUB) + b * BATCH
        sb = wid * nsub_pt + b * sub_pb
        pltpu.sync_copy(col2.at[pl.ds(sb, sub_pb)], colb)
        pltpu.sync_copy(ew.at[pl.ds(eb, BATCH)], ewb)
        for j in range(sub_pb):
            pltpu.sync_copy(ewb.at[pl.ds(j * SUB, SUB)],
                            dacc.at[colb.at[j]], add=True)

    plsc.subcore_barrier()
    pltpu.sync_copy(dacc.at[pl.ds(sid * npt, npt)],
                    out.at[cid].at[pl.ds(sid * npt, npt)])


def _sc_deg(col2, ew, np_, ep):
    nsub_pt = ep // (NCORES * NTILES * SUB)
    mesh = plsc.VectorSubcoreMesh(core_axis_name="c", subcore_axis_name="s")
    return pl.kernel(
        functools.partial(_sc_deg_body, np_, nsub_pt),
        out_type=jax.ShapeDtypeStruct((NCORES, np_), jnp.float32),
        mesh=mesh,
        compiler_params=pltpu.CompilerParams(use_tc_tiling_on_sc=False),
        scratch_types=[
            pltpu.VMEM_SHARED((np_,), jnp.float32),
            pltpu.VMEM((BATCH // SUB, SUB), jnp.int32),
            pltpu.VMEM((BATCH,), jnp.float32),
            pltpu.VMEM((np_ // NTILES,), jnp.float32),
        ],
    )(col2, ew)


GRP = 512
NSLOT = 4


def _sc_edge_body(nchunks, np_, nsub_pt, zn,
                  y, row, col2, ew, out,
                  acc, rowg, colg, ewg, rows, zb, zcol,
                  lsem0, lsem1, gsem0, gsem1, ssem0, ssem1):
    cid = lax.axis_index("c")
    sid = lax.axis_index("s")
    nprt = np_ // NTILES
    spg = GRP // SUB
    ng = (nsub_pt * SUB) // GRP
    wid = cid * NTILES + sid
    tile_e = wid * (nsub_pt * SUB)
    tile_s = wid * nsub_pt
    lsem = (lsem0, lsem1)
    gsem = (gsem0, gsem1)
    ssem = (ssem0, ssem1)

    @pl.loop(0, zn)
    def _(i):
        zb[i] = jnp.zeros((LANES,), jnp.float32)

    @pl.loop(0, spg)
    def _(i):
        for j in range(SUB // LANES):
            zcol[i, pl.ds(j * LANES, LANES)] = jnp.zeros((LANES,), jnp.int32)

    def fire_linear(p, s, g):
        pltpu.async_copy(row.at[pl.ds(tile_e + g * GRP, GRP)],
                         rowg.at[p], lsem[p])
        pltpu.async_copy(ew.at[pl.ds(tile_e + g * GRP, GRP)],
                         ewg.at[p], lsem[p])
        pltpu.async_copy(col2.at[pl.ds(tile_s + g * spg, spg)],
                         colg.at[s], lsem[p])

    def wait_linear(p, s, g):
        pltpu.make_async_copy(row.at[pl.ds(tile_e + g * GRP, GRP)],
                              rowg.at[p], lsem[p]).wait()
        pltpu.make_async_copy(ew.at[pl.ds(tile_e + g * GRP, GRP)],
                              ewg.at[p], lsem[p]).wait()
        pltpu.make_async_copy(col2.at[pl.ds(tile_s + g * spg, spg)],
                              colg.at[s], lsem[p]).wait()

    def fire_gather(c, p):
        for j in range(spg):
            pltpu.async_copy(
                y.at[c].at[rowg.at[p].at[pl.ds(j * SUB, SUB)]],
                rows.at[p].at[pl.ds(j * SUB, SUB)], gsem[p])

    def wait_gather(c, p):
        for j in range(spg):
            pltpu.make_async_copy(
                y.at[c].at[rowg.at[p].at[pl.ds(j * SUB, SUB)]],
                rows.at[p].at[pl.ds(j * SUB, SUB)], gsem[p]).wait()

    def fire_scatter(p, s):
        for j in range(spg):
            pltpu.async_copy(rows.at[p].at[pl.ds(j * SUB, SUB)],
                             acc.at[colg.at[s].at[j]], ssem[p], add=True)

    def wait_scatter(p, s):
        for j in range(spg):
            pltpu.make_async_copy(rows.at[p].at[pl.ds(j * SUB, SUB)],
                                  acc.at[colg.at[s].at[j]],
                                  ssem[p]).wait()

    def fire_prime(p):
        for j in range(spg):
            pltpu.async_copy(rows.at[p].at[pl.ds(j * SUB, SUB)],
                             acc.at[zcol.at[j]], ssem[p], add=True)

    def wait_prime(p):
        for j in range(spg):
            pltpu.make_async_copy(rows.at[p].at[pl.ds(j * SUB, SUB)],
                                  acc.at[zcol.at[j]], ssem[p]).wait()

    def scale(p):
        @plsc.parallel_loop(0, GRP // LANES, unroll=2)
        def _(g):
            e0 = g * LANES
            ewv = ewg[p, pl.ds(e0, LANES)]
            for j in range(LANES):
                w16 = jnp.full((LANES,), ewv[j], jnp.float32)
                rows[p, e0 + j] = rows[p, e0 + j] * w16

    for c in range(nchunks):
        @pl.loop(0, nprt // zn)
        def _(i):
            pltpu.sync_copy(zb, acc.at[pl.ds(sid * nprt + i * zn, zn)])

        plsc.subcore_barrier()

        @pl.loop(0, GRP)
        def _(i):
            rows[1, i] = jnp.zeros((LANES,), jnp.float32)

        fire_prime(1)
        pltpu.sync_copy(row.at[pl.ds(tile_e, GRP)], rowg.at[0])
        pltpu.sync_copy(ew.at[pl.ds(tile_e, GRP)], ewg.at[0])
        pltpu.sync_copy(col2.at[pl.ds(tile_s, spg)], colg.at[0])
        fire_gather(c, 0)
        fire_linear(1, 1, 1)

        @pl.loop(0, ng // NSLOT)
        def _(i):
            for u in range(NSLOT):
                g = i * NSLOT + u
                p = u % 2
                q = 1 - p
                wait_gather(c, p)
                wait_linear(q, (u + 1) % NSLOT, g + 1)
                wait_scatter(q, (u + 3) % NSLOT)
                fire_gather(c, q)
                scale(p)
                fire_scatter(p, u % NSLOT)
                fire_linear(p, (u + 2) % NSLOT, g + 2)

        wait_gather(c, 0)
        wait_linear(1, (ng + 1) % NSLOT, ng + 1)
        wait_scatter(1, (ng - 1) % NSLOT)

        plsc.subcore_barrier()
        pltpu.sync_copy(acc.at[pl.ds(sid * nprt, nprt)],
                        out.at[cid].at[c].at[pl.ds(sid * nprt, nprt)])
        plsc.subcore_barrier()


def _sc_edge(y, row, col2, ew, np_, ep):
    nchunks = y.shape[0]
    nsub_pt = ep // (NCORES * NTILES * SUB)
    zn = 98
    mesh = plsc.VectorSubcoreMesh(core_axis_name="c", subcore_axis_name="s")
    return pl.kernel(
        functools.partial(_sc_edge_body, nchunks, np_, nsub_pt, zn),
        out_type=jax.ShapeDtypeStruct((NCORES, nchunks, np_, LANES),
                                      jnp.float32),
        mesh=mesh,
        compiler_params=pltpu.CompilerParams(use_tc_tiling_on_sc=False),
        scratch_types=[
            pltpu.VMEM_SHARED((np_, LANES), jnp.float32),
            pltpu.VMEM((2, GRP), jnp.int32),
            pltpu.VMEM((NSLOT, GRP // SUB, SUB), jnp.int32),
            pltpu.VMEM((2, GRP), jnp.float32),
            pltpu.VMEM((2, GRP, LANES), jnp.float32),
            pltpu.VMEM((zn, LANES), jnp.float32),
            pltpu.VMEM((GRP // SUB, SUB), jnp.int32),
            pltpu.SemaphoreType.DMA,
            pltpu.SemaphoreType.DMA,
            pltpu.SemaphoreType.DMA,
            pltpu.SemaphoreType.DMA,
            pltpu.SemaphoreType.DMA,
            pltpu.SemaphoreType.DMA,
        ],
    )(y, row, col2, ew)


def _tc_prep(degp, x, w1p, np_):
    nblk = np_ // BN
    fin = x.shape[1]

    def body(degp_ref, x_ref, w_ref, dinv_ref, y_ref):
        deg = degp_ref[0] + degp_ref[1] + 1.0
        dinv = jnp.where(deg > 0, lax.rsqrt(deg), 0.0)
        dinv_ref[...] = dinv[:, None]
        xw = jnp.dot(x_ref[...], w_ref[...],
                     preferred_element_type=jnp.float32)
        y_ref[0] = xw * dinv[:, None]

    return pl.pallas_call(
        body,
        grid=(nblk,),
        in_specs=[
            pl.BlockSpec((NCORES, BN), lambda i: (0, i)),
            pl.BlockSpec((BN, fin), lambda i: (i, 0)),
            pl.BlockSpec(w1p.shape, lambda i: (0, 0)),
        ],
        out_specs=[
            pl.BlockSpec((BN, 1), lambda i: (i, 0)),
            pl.BlockSpec((1, BN, LANES), lambda i: (0, i, 0)),
        ],
        out_shape=[
            jax.ShapeDtypeStruct((np_, 1), jnp.float32),
            jax.ShapeDtypeStruct((1, np_, LANES), jnp.float32),
        ],
    )(degp, x, w1p)


def _tc_boundary(S, y, dinv, bp, wn, np_):
    kin = y.shape[0]
    kout = wn.shape[1] // LANES
    nblk = np_ // BN

    def body(s_ref, y_ref, d_ref, b_ref, w_ref, o_ref):
        tot = s_ref[0] + s_ref[1] + y_ref[...]
        z = jnp.concatenate([tot[c] for c in range(kin)], axis=1)
        dv = d_ref[...]
        h = _leaky(z * dv + b_ref[...])
        yn = jnp.dot(h, w_ref[...], preferred_element_type=jnp.float32) * dv
        for c in range(kout):
            o_ref[c] = yn[:, c * LANES:(c + 1) * LANES]

    return pl.pallas_call(
        body,
        grid=(nblk,),
        in_specs=[
            pl.BlockSpec((NCORES, kin, BN, LANES), lambda i: (0, 0, i, 0)),
            pl.BlockSpec((kin, BN, LANES), lambda i: (0, i, 0)),
            pl.BlockSpec((BN, 1), lambda i: (i, 0)),
            pl.BlockSpec(bp.shape, lambda i: (0, 0)),
            pl.BlockSpec(wn.shape, lambda i: (0, 0)),
        ],
        out_specs=pl.BlockSpec((kout, BN, LANES), lambda i: (0, i, 0)),
        out_shape=jax.ShapeDtypeStruct((kout, np_, LANES), jnp.float32),
    )(S, y, dinv, bp, wn)


def _tc_final(S, y, dinv, bp, bat3, wf1, cf1, wf2, cf2, wf3, cf3,
              np_, ngraphs):
    kin = y.shape[0]
    w = kin * LANES
    nblk = np_ // BN

    def body(s_ref, y_ref, d_ref, b_ref, bat_ref,
             w1_ref, c1_ref, w2_ref, c2_ref, w3_ref, c3_ref, o_ref, pacc):
        i = pl.program_id(0)

        @pl.when(i == 0)
        def _():
            pacc[...] = jnp.zeros_like(pacc)

        tot = s_ref[0] + s_ref[1] + y_ref[...]
        z = jnp.concatenate([tot[c] for c in range(kin)], axis=1)
        h = _leaky(z * d_ref[...] + b_ref[...])
        bi = bat_ref[0, 0, :]
        gid = lax.broadcasted_iota(jnp.int32, (ngraphs, BN), 0)
        onehot = (gid == bi[None, :]).astype(jnp.float32)
        pacc[...] += jnp.dot(onehot, h, preferred_element_type=jnp.float32)

        @pl.when(i == nblk - 1)
        def _():
            p = pacc[...]
            x1 = _leaky(jnp.dot(p, w1_ref[...],
                                preferred_element_type=jnp.float32)
                        + c1_ref[...])
            x2 = _leaky(jnp.dot(x1, w2_ref[...],
                                preferred_element_type=jnp.float32)
                        + c2_ref[...])
            o_ref[...] = jnp.dot(x2, w3_ref[...],
                                 preferred_element_type=jnp.float32) \
                + c3_ref[...]

    return pl.pallas_call(
        body,
        grid=(nblk,),
        in_specs=[
            pl.BlockSpec((NCORES, kin, BN, LANES), lambda i: (0, 0, i, 0)),
            pl.BlockSpec((kin, BN, LANES), lambda i: (0, i, 0)),
            pl.BlockSpec((BN, 1), lambda i: (i, 0)),
            pl.BlockSpec(bp.shape, lambda i: (0, 0)),
            pl.BlockSpec((1, 1, BN), lambda i: (i, 0, 0)),
            pl.BlockSpec(wf1.shape, lambda i: (0, 0)),
            pl.BlockSpec(cf1.shape, lambda i: (0, 0)),
            pl.BlockSpec(wf2.shape, lambda i: (0, 0)),
            pl.BlockSpec(cf2.shape, lambda i: (0, 0)),
            pl.BlockSpec(wf3.shape, lambda i: (0, 0)),
            pl.BlockSpec(cf3.shape, lambda i: (0, 0)),
        ],
        out_specs=pl.BlockSpec((ngraphs, wf3.shape[1]), lambda i: (0, 0)),
        out_shape=jax.ShapeDtypeStruct((ngraphs, wf3.shape[1]), jnp.float32),
        scratch_shapes=[pltpu.VMEM((ngraphs, w), jnp.float32)],
    )(S, y, dinv, bp, bat3, wf1, cf1, wf2, cf2, wf3, cf3)


def _padw(w, b):
    fin, fout = w.shape
    fin_p = -(-fin // LANES) * LANES if fin > 4 else fin
    fout_p = -(-fout // LANES) * LANES
    wp = jnp.pad(w, ((0, fin_p - fin), (0, fout_p - fout)))
    bp = jnp.pad(b, (0, fout_p - fout)).reshape(1, fout_p)
    return wp, bp


def kernel(node_features, edge_index, edge_weight, batch_index,
           W1, b1, W2, b2, W3, b3, Wf1, bf1, Wf2, bf2, Wf3, bf3):
    n0 = node_features.shape[0]
    e0 = edge_index.shape[1]
    ngraphs = 64

    np_ = -(-n0 // BN) * BN
    epb = NCORES * NTILES * BATCH
    ep = -(-e0 // epb) * epb

    x = jnp.pad(node_features, ((0, np_ - n0), (0, 0)))
    pe = 2 * GRP
    row = jnp.pad(edge_index[0], (0, ep + pe - e0))
    col = jnp.pad(edge_index[1], (0, ep + pe - e0))
    col2 = col.reshape((ep + pe) // SUB, SUB)
    ew = jnp.pad(edge_weight, (0, ep + pe - e0))
    bat = jnp.pad(batch_index, (0, np_ - n0), constant_values=ngraphs)
    bat3 = bat.reshape(np_ // BN, 1, BN)

    w1p, b1p = _padw(W1, b1)
    w2p, b2p = _padw(W2, b2)
    w3p, b3p = _padw(W3, b3)
    cf1 = bf1.reshape(1, -1)
    cf2 = bf2.reshape(1, -1)
    cf3 = bf3.reshape(1, -1)

    degp = _sc_deg(col2, ew, np_, ep)
    dinv, y1 = _tc_prep(degp, x, w1p, np_)
    s1 = _sc_edge(y1, row, col2, ew, np_, ep)
    y2 = _tc_boundary(s1, y1, dinv, b1p, w2p, np_)
    s2 = _sc_edge(y2, row, col2, ew, np_, ep)
    y3 = _tc_boundary(s2, y2, dinv, b2p, w3p, np_)
    s3 = _sc_edge(y3, row, col2, ew, np_, ep)
    return _tc_final(s3, y3, dinv, b3p, bat3,
                     Wf1, cf1, Wf2, cf2, Wf3, cf3, np_, ngraphs)
